# baseline (speedup 1.0000x reference)
import functools
import jax
import jax.numpy as jnp
from jax import lax
from jax.experimental import pallas as pl
from jax.experimental.pallas import tpu as pltpu
from jax.experimental.pallas import tpu_sc as plsc

CHUNK = 16


def kernel(x, pos_table):
    batch, seq_len, d_model = x.shape
    info = plsc.get_sparse_core_info()
    nc, ns, nl = info.num_cores, info.num_subcores, info.num_lanes
    nw = nc * ns
    rows_per_w = seq_len // nw
    n_chunks = rows_per_w // CHUNK
    lanes_per_row = d_model // nl

    mesh = plsc.VectorSubcoreMesh(core_axis_name="c", subcore_axis_name="s")

    @functools.partial(
        pl.kernel,
        mesh=mesh,
        out_type=jax.ShapeDtypeStruct(x.shape, x.dtype),
        scratch_types=[
            pltpu.VMEM((CHUNK, d_model), jnp.float32),
            pltpu.VMEM((CHUNK, d_model), jnp.float32),
        ],
    )
    def k(x_hbm, pos_hbm, out_hbm, pos_v, xb_v):
        wid = lax.axis_index("s") * nc + lax.axis_index("c")
        s_base = wid * rows_per_w

        def chunk_body(ci, _):
            row0 = s_base + ci * CHUNK
            pltpu.sync_copy(pos_hbm.at[pl.ds(row0, CHUNK)], pos_v)
            for b in range(batch):
                pltpu.sync_copy(x_hbm.at[b, pl.ds(row0, CHUNK)], xb_v)

                def add_row(r, _):
                    def add_vec(j, _):
                        sl = pl.ds(j * nl, nl)
                        xb_v[r, sl] = xb_v[r, sl] + pos_v[r, sl]
                        return 0

                    return lax.fori_loop(0, lanes_per_row, add_vec, 0, unroll=8)

                lax.fori_loop(0, CHUNK, add_row, 0)
                pltpu.sync_copy(xb_v, out_hbm.at[b, pl.ds(row0, CHUNK)])
            return 0

        lax.fori_loop(0, n_chunks, chunk_body, 0)

    return k(x, pos_table)


if __name__ == "__main__":
    pass


# speedup vs baseline: 1.4026x; 1.4026x over previous
import functools
import jax
import jax.numpy as jnp
from jax import lax
from jax.experimental import pallas as pl
from jax.experimental.pallas import tpu as pltpu
from jax.experimental.pallas import tpu_sc as plsc

CHUNK = 16


def kernel(x, pos_table):
    batch, seq_len, d_model = x.shape
    info = plsc.get_sparse_core_info()
    nc, ns, nl = info.num_cores, info.num_subcores, info.num_lanes
    nw = nc * ns
    rows_per_w = seq_len // nw
    n_chunks = rows_per_w // CHUNK
    vecs_per_row = d_model // nl
    n_steps = n_chunks * batch

    mesh = plsc.VectorSubcoreMesh(core_axis_name="c", subcore_axis_name="s")

    @functools.partial(
        pl.kernel,
        mesh=mesh,
        out_type=jax.ShapeDtypeStruct(x.shape, x.dtype),
        scratch_types=[
            pltpu.VMEM((CHUNK, d_model), jnp.float32),
            pltpu.VMEM((CHUNK, d_model), jnp.float32),
            pltpu.VMEM((CHUNK, d_model), jnp.float32),
            pltpu.VMEM((CHUNK, d_model), jnp.float32),
            pltpu.VMEM((CHUNK, d_model), jnp.float32),
            pltpu.SemaphoreType.DMA,
            pltpu.SemaphoreType.DMA,
            pltpu.SemaphoreType.DMA,
            pltpu.SemaphoreType.DMA,
            pltpu.SemaphoreType.DMA,
            pltpu.SemaphoreType.DMA,
            pltpu.SemaphoreType.DMA,
            pltpu.SemaphoreType.DMA,
        ],
    )
    def k(x_hbm, pos_hbm, out_hbm, p0, p1, xb0, xb1, xb2,
          ps0, ps1, is0, is1, is2, os0, os1, os2):
        wid = lax.axis_index("s") * nc + lax.axis_index("c")
        s_base = wid * rows_per_w
        pos_bufs, pos_sems = [p0, p1], [ps0, ps1]
        x_bufs, in_sems, out_sems = [xb0, xb1, xb2], [is0, is1, is2], [os0, os1, os2]

        def pos_in(ci):
            pb = ci % 2
            return pltpu.async_copy(
                pos_hbm.at[pl.ds(s_base + ci * CHUNK, CHUNK)],
                pos_bufs[pb], pos_sems[pb])

        def x_in(k_step):
            ci, b = divmod(k_step, batch)
            xb = k_step % 3
            return pltpu.async_copy(
                x_hbm.at[b, pl.ds(s_base + ci * CHUNK, CHUNK)],
                x_bufs[xb], in_sems[xb])

        def x_out(k_step):
            ci, b = divmod(k_step, batch)
            xb = k_step % 3
            return pltpu.async_copy(
                x_bufs[xb],
                out_hbm.at[b, pl.ds(s_base + ci * CHUNK, CHUNK)],
                out_sems[xb])

        pos_pending = {0: pos_in(0), 1: pos_in(1)}
        in_pending = {0: x_in(0), 1: x_in(1)}
        out_pending = {}

        for k_step in range(n_steps):
            ci, b = divmod(k_step, batch)
            xb = k_step % 3
            pb = ci % 2
            in_pending.pop(xb).wait()
            if b == 0:
                pos_pending.pop(pb).wait()
            xv, pv = x_bufs[xb], pos_bufs[pb]

            def add_row(r, _):
                def add_vec(j, _):
                    sl = pl.ds(j * nl, nl)
                    xv[r, sl] = xv[r, sl] + pv[r, sl]
                    return 0

                return lax.fori_loop(0, vecs_per_row, add_vec, 0, unroll=16)

            lax.fori_loop(0, CHUNK, add_row, 0)

            out_pending[xb] = x_out(k_step)

            nxt = k_step + 2
            if nxt < n_steps:
                nb = nxt % 3
                if nb in out_pending:
                    out_pending.pop(nb).wait()
                in_pending[nb] = x_in(nxt)
            if b == batch - 1 and ci + 2 < n_chunks:
                pos_pending[(ci + 2) % 2] = pos_in(ci + 2)

        for h in out_pending.values():
            h.wait()

    return k(x, pos_table)


# speedup vs baseline: 1.4345x; 1.0228x over previous
import functools
import jax
import jax.numpy as jnp
from jax import lax
from jax.experimental import pallas as pl
from jax.experimental.pallas import tpu as pltpu
from jax.experimental.pallas import tpu_sc as plsc

CHUNK = 16


def kernel(x, pos_table):
    batch, seq_len, d_model = x.shape
    info = plsc.get_sparse_core_info()
    nc, ns, nl = info.num_cores, info.num_subcores, info.num_lanes
    nw = nc * ns
    rows_per_w = seq_len // nw
    n_chunks = rows_per_w // CHUNK
    vecs_per_row = d_model // nl
    n_steps = n_chunks * batch
    spb = 2 * batch

    mesh = plsc.VectorSubcoreMesh(core_axis_name="c", subcore_axis_name="s")

    @functools.partial(
        pl.kernel,
        mesh=mesh,
        out_type=jax.ShapeDtypeStruct(x.shape, x.dtype),
        scratch_types=[
            pltpu.VMEM((CHUNK, d_model), jnp.float32),
            pltpu.VMEM((CHUNK, d_model), jnp.float32),
            pltpu.VMEM((CHUNK, d_model), jnp.float32),
            pltpu.VMEM((CHUNK, d_model), jnp.float32),
            pltpu.VMEM((CHUNK, d_model), jnp.float32),
            pltpu.VMEM((CHUNK, d_model), jnp.float32),
            pltpu.SemaphoreType.DMA,
            pltpu.SemaphoreType.DMA,
            pltpu.SemaphoreType.DMA,
            pltpu.SemaphoreType.DMA,
            pltpu.SemaphoreType.DMA,
            pltpu.SemaphoreType.DMA,
            pltpu.SemaphoreType.DMA,
            pltpu.SemaphoreType.DMA,
            pltpu.SemaphoreType.DMA,
            pltpu.SemaphoreType.DMA,
        ],
    )
    def k(x_hbm, pos_hbm, out_hbm, p0, p1, xb0, xb1, xb2, xb3,
          ps0, ps1, is0, is1, is2, is3, os0, os1, os2, os3):
        wid = lax.axis_index("s") * nc + lax.axis_index("c")
        s_base = wid * rows_per_w
        pos_bufs, pos_sems = [p0, p1], [ps0, ps1]
        x_bufs = [xb0, xb1, xb2, xb3]
        in_sems = [is0, is1, is2, is3]
        out_sems = [os0, os1, os2, os3]

        def issue_pos(ci):
            pass

        def pos_start(ci, pb):
            pltpu.async_copy(
                pos_hbm.at[pl.ds(s_base + ci * CHUNK, CHUNK)],
                pos_bufs[pb], pos_sems[pb])

        def in_start(ci, b, xb):
            pltpu.async_copy(
                x_hbm.at[b, pl.ds(s_base + ci * CHUNK, CHUNK)],
                x_bufs[xb], in_sems[xb])

        def out_start(ci, b, xb):
            pltpu.async_copy(
                x_bufs[xb],
                out_hbm.at[b, pl.ds(s_base + ci * CHUNK, CHUNK)],
                out_sems[xb])

        def wait_in(xb):
            pltpu.make_async_copy(
                x_hbm.at[0, pl.ds(0, CHUNK)], x_bufs[xb], in_sems[xb]).wait()

        def wait_pos(pb):
            pltpu.make_async_copy(
                pos_hbm.at[pl.ds(0, CHUNK)], pos_bufs[pb], pos_sems[pb]).wait()

        def wait_out(xb):
            pltpu.make_async_copy(
                x_bufs[xb], out_hbm.at[0, pl.ds(0, CHUNK)],
                out_sems[xb]).wait()

        def add_chunk(xv, pv):
            def add_row(r, _):
                def add_vec(j, _):
                    sl = pl.ds(j * nl, nl)
                    xv[r, sl] = xv[r, sl] + pv[r, sl]
                    return 0
                return lax.fori_loop(0, vecs_per_row, add_vec, 0, unroll=16)
            lax.fori_loop(0, CHUNK, add_row, 0)

        def steady_step(ci_base, i, first_step=False, last_body=False):
            ci = ci_base + i // batch
            b = i % batch
            xb = i % 4
            pb = (i // batch) % 2
            wait_in(xb)
            if b == 0:
                wait_pos(pb)
            add_chunk(x_bufs[xb], pos_bufs[pb])
            out_start(ci, b, xb)
            i3 = i + 3
            xb_t = i3 % 4
            b_t = i3 % batch
            ci_t = ci_base + i3 // batch
            if not first_step:
                wait_out(xb_t)
            in_start(ci_t, b_t, xb_t)
            if b == batch - 1 and not last_body:
                pos_start(ci + 2, pb)

        in_start(0, 0, 0)
        in_start(0, 1, 1)
        in_start(0, 2, 2)
        pos_start(0, 0)
        pos_start(1, 1)
        for i in range(spb):
            steady_step(0, i, first_step=(i == 0))

        def body(it, _):
            ci_base = it * 2
            for i in range(spb):
                steady_step(ci_base, i)
            return 0
        lax.fori_loop(1, n_steps // spb - 1, body, 0)

        for i in range(spb):
            ci = n_chunks - 2 + i // batch
            b = i % batch
            xb = i % 4
            pb = (i // batch) % 2
            wait_in(xb)
            if b == 0:
                wait_pos(pb)
            add_chunk(x_bufs[xb], pos_bufs[pb])
            out_start(ci, b, xb)
            i3 = i + 3
            if i3 < spb + 3 and (n_steps - spb) + i3 < n_steps:
                xb_t = i3 % 4
                wait_out(xb_t)
                in_start(n_chunks - 2 + i3 // batch, i3 % batch, xb_t)

        for xb in range(4):
            wait_out(xb)

    return k(x, pos_table)


# speedup vs baseline: 5.1984x; 3.6239x over previous
import jax
import jax.numpy as jnp
from jax.experimental import pallas as pl

BLOCK_S = 512


def _add_body(x_ref, pos_ref, o_ref):
    o_ref[...] = x_ref[...] + pos_ref[...][None]


def kernel(x, pos_table):
    batch, seq_len, d_model = x.shape
    grid = (seq_len // BLOCK_S,)
    return pl.pallas_call(
        _add_body,
        grid=grid,
        in_specs=[
            pl.BlockSpec((batch, BLOCK_S, d_model), lambda s: (0, s, 0)),
            pl.BlockSpec((BLOCK_S, d_model), lambda s: (s, 0)),
        ],
        out_specs=pl.BlockSpec((batch, BLOCK_S, d_model), lambda s: (0, s, 0)),
        out_shape=jax.ShapeDtypeStruct(x.shape, x.dtype),
    )(x, pos_table)


# speedup vs baseline: 5.2286x; 1.0058x over previous
import jax
import jax.numpy as jnp
from jax.experimental import pallas as pl

BLOCK_S = 2048


def _add_body(x_ref, pos_ref, o_ref):
    o_ref[...] = x_ref[...] + pos_ref[...]


def kernel(x, pos_table):
    batch, seq_len, d_model = x.shape
    grid = (seq_len // BLOCK_S, batch)
    return pl.pallas_call(
        _add_body,
        grid=grid,
        in_specs=[
            pl.BlockSpec((None, BLOCK_S, d_model), lambda s, b: (b, s, 0)),
            pl.BlockSpec((BLOCK_S, d_model), lambda s, b: (s, 0)),
        ],
        out_specs=pl.BlockSpec((None, BLOCK_S, d_model), lambda s, b: (b, s, 0)),
        out_shape=jax.ShapeDtypeStruct(x.shape, x.dtype),
    )(x, pos_table)
